# baseline (speedup 1.0000x reference)
import functools

import jax
import jax.numpy as jnp
from jax import lax
from jax.experimental import pallas as pl
from jax.experimental.pallas import tpu as pltpu
from jax.experimental.pallas import tpu_sc as plsc

N = 10000
D = 128
NC = 2
NS = 16
NW = NC * NS
B = 32
NP = 10240
ROWS_PER_TILE = NP // NS
DEN_R = NP // 128


def _norm_body(x_ref, xn_ref, nb_ref):
    xb = x_ref[...]
    ss = jnp.sum(xb * xb, axis=1, keepdims=True)
    nrm = jnp.maximum(jnp.sqrt(ss), 1e-12)
    xn_ref[...] = xb / nrm
    nb_ref[...] = jnp.broadcast_to(nrm, xb.shape)


def _final_body(a0_ref, a1_ref, dt_ref, o_ref):
    den = jnp.sum(dt_ref[...], axis=1, keepdims=True)
    s = a0_ref[0] + a1_ref[0]
    o_ref[...] = jnp.maximum(s / jnp.maximum(den, 1e-16), 0.0)


def _make_edge_kernel(e_full, nb0, nb1):
    mesh = plsc.VectorSubcoreMesh(
        core_axis_name="c", subcore_axis_name="s", num_cores=NC, num_subcores=NS
    )

    @functools.partial(
        pl.kernel,
        out_type=(
            jax.ShapeDtypeStruct((NC, NP, D), jnp.float32),
            jax.ShapeDtypeStruct((NC, DEN_R, 128), jnp.float32),
        ),
        mesh=mesh,
        compiler_params=pltpu.CompilerParams(needs_layout_passes=False),
        scratch_types=[
            pltpu.VMEM((N,), jnp.float32),
            pltpu.VMEM((DEN_R, 128), jnp.float32),
            pltpu.VMEM((3, B), jnp.int32),
            pltpu.VMEM((3, B), jnp.int32),
            pltpu.VMEM((3, B), jnp.int32),
            pltpu.VMEM((3, B, D), jnp.float32),
            pltpu.VMEM((3, B, D), jnp.float32),
            pltpu.VMEM((16,), jnp.float32),
            pltpu.VMEM((DEN_R,), jnp.int32),
            pltpu.SemaphoreType.DMA((3,)),
            pltpu.SemaphoreType.DMA((3,)),
            pltpu.SemaphoreType.DMA((3,)),
            pltpu.VMEM_SHARED((NP, D), jnp.float32),
            pltpu.VMEM_SHARED((DEN_R, 128), jnp.float32),
        ],
    )
    def edge_kernel(xn_hbm, n_hbm, src_hbm, dst_hbm, beta_hbm,
                    acc_out, den_out,
                    ntab, denom_v, sidx, didx, sdidx, srows, drows,
                    beta_v, iden, isem, gsem, ssem, acc_sh, dden_sh):
        cid = lax.axis_index("c")
        sid = lax.axis_index("s")
        n_my = jnp.where(cid == 0, nb0, nb1)
        tile_start = jnp.where(cid == 0, sid * (nb0 * B),
                               NS * (nb0 * B) + sid * (nb1 * B))

        pltpu.sync_copy(n_hbm, ntab)
        pltpu.sync_copy(beta_hbm, beta_v)
        bvec = beta_v[pl.ds(0, 16)]

        zero16 = jnp.zeros((16,), jnp.float32)

        def zero_den(i, carry):
            for s in range(8):
                denom_v[i, pl.ds(s * 16, 16)] = zero16
            return carry

        lax.fori_loop(0, DEN_R, zero_den, 0)

        def zero_sr(i, carry):
            for s in range(8):
                srows[0, i, pl.ds(s * 16, 16)] = zero16
            return carry

        lax.fori_loop(0, B, zero_sr, 0)

        lane = lax.iota(jnp.int32, 16)
        for q in range(DEN_R // 16):
            iden[pl.ds(q * 16, 16)] = lane + q * 16

        row0 = sid * ROWS_PER_TILE
        for q in range(ROWS_PER_TILE // B):
            pltpu.sync_copy(srows.at[0], acc_sh.at[pl.ds(row0 + q * B, B)])

        @pl.when(sid == 0)
        def _():
            pltpu.sync_copy(srows.at[0], dden_sh.at[pl.ds(0, B)])
            pltpu.sync_copy(srows.at[0], dden_sh.at[pl.ds(B, B)])
            pltpu.sync_copy(srows.at[0, pl.ds(0, DEN_R - 2 * B)],
                            dden_sh.at[pl.ds(2 * B, DEN_R - 2 * B)])

        plsc.subcore_barrier()

        def idx_start(slot, blk_id):
            base = tile_start + blk_id * B
            pltpu.async_copy(src_hbm.at[pl.ds(base, B)], sidx.at[slot],
                             isem.at[slot])
            pltpu.async_copy(dst_hbm.at[pl.ds(base, B)], didx.at[slot],
                             isem.at[slot])

        def idx_wait(slot, blk_id):
            base = tile_start + blk_id * B
            pltpu.make_async_copy(src_hbm.at[pl.ds(base, B)], sidx.at[slot],
                                  isem.at[slot]).wait()
            pltpu.make_async_copy(dst_hbm.at[pl.ds(base, B)], didx.at[slot],
                                  isem.at[slot]).wait()

        def gather_start(slot):
            pltpu.async_copy(xn_hbm.at[sidx.at[slot]], srows.at[slot],
                             gsem.at[slot])
            pltpu.async_copy(xn_hbm.at[didx.at[slot]], drows.at[slot],
                             gsem.at[slot])

        def gather_wait(slot):
            pltpu.make_async_copy(xn_hbm.at[sidx.at[slot]], srows.at[slot],
                                  gsem.at[slot]).wait()
            pltpu.make_async_copy(xn_hbm.at[didx.at[slot]], drows.at[slot],
                                  gsem.at[slot]).wait()

        def scatter_start(slot):
            pltpu.async_copy(srows.at[slot], acc_sh.at[sdidx.at[slot]],
                             ssem.at[slot], add=True)

        def scatter_wait(slot):
            pltpu.make_async_copy(srows.at[slot], acc_sh.at[sdidx.at[slot]],
                                  ssem.at[slot]).wait()

        def compute_block(slot, g):
            base = tile_start + g * B

            def grp(k, c2):
                sl = pl.ds(k * 16, 16)
                si = sidx[slot, sl]
                di = didx[slot, sl]
                d16 = jnp.zeros((16,), jnp.float32)
                for j in range(16):
                    e = k * 16 + j
                    acc = (srows[slot, e, pl.ds(0, 16)]
                           * drows[slot, e, pl.ds(0, 16)])
                    for s in range(1, 8):
                        fsl = pl.ds(s * 16, 16)
                        acc = acc + srows[slot, e, fsl] * drows[slot, e, fsl]
                    d16 = jnp.where(lane == j, jnp.sum(acc), d16)
                gid = base + k * 16 + lane
                w = jnp.where(gid < e_full, jnp.exp(bvec * d16), 0.0)
                plsc.addupdate_scatter(denom_v, [di // 128, di % 128], w)
                n_s = plsc.load_gather(ntab, [si])
                wn = w * n_s
                sdidx[slot, sl] = di
                for j in range(16):
                    e = k * 16 + j
                    wj = wn[j]
                    for s in range(8):
                        fsl = pl.ds(s * 16, 16)
                        srows[slot, e, fsl] = srows[slot, e, fsl] * wj
                return c2

            lax.fori_loop(0, B // 16, grp, 0)

        idx_start(0, 0)
        idx_start(1, 1)
        idx_wait(0, 0)
        gather_start(0)

        def blk(g, carry):
            slot = lax.rem(g, 3)
            sn = lax.rem(g + 1, 3)
            sp = lax.rem(g + 2, 3)

            @pl.when(g >= 2)
            def _():
                scatter_wait(sn)

            @pl.when(g + 1 < n_my)
            def _():
                idx_wait(sn, g + 1)
                gather_start(sn)

            @pl.when(g + 2 < n_my)
            def _():
                idx_start(sp, g + 2)

            gather_wait(slot)
            compute_block(slot, g)
            scatter_start(slot)
            return carry

        lax.fori_loop(0, n_my, blk, 0)
        scatter_wait(lax.rem(n_my - 2, 3))
        scatter_wait(lax.rem(n_my - 1, 3))

        pltpu.sync_copy(denom_v, dden_sh.at[iden], add=True)
        plsc.subcore_barrier()
        pltpu.sync_copy(acc_sh.at[pl.ds(row0, ROWS_PER_TILE)],
                        acc_out.at[cid, pl.ds(row0, ROWS_PER_TILE)])

        @pl.when(sid == 0)
        def _():
            pltpu.sync_copy(dden_sh, den_out.at[cid])

    return edge_kernel


def kernel(x, edge_index, beta):
    src = edge_index[0].astype(jnp.int32)
    dst = edge_index[1].astype(jnp.int32)
    loop = jnp.arange(N, dtype=jnp.int32)
    e_full = src.shape[0] + N
    nb_tot = -(-e_full // (NS * B))
    nb0 = int(nb_tot * 0.51)
    nb1 = nb_tot - nb0
    e_pad = NS * B * (nb0 + nb1)
    pad = e_pad - e_full
    src_full = jnp.concatenate([src, loop, jnp.zeros((pad,), jnp.int32)])
    dst_full = jnp.concatenate([dst, loop, jnp.zeros((pad,), jnp.int32)])
    beta16 = jnp.broadcast_to(beta.astype(jnp.float32), (16,))

    grid_r = 10
    rb = N // grid_r
    xn, nb = pl.pallas_call(
        _norm_body,
        grid=(grid_r,),
        in_specs=[pl.BlockSpec((rb, D), lambda i: (i, 0))],
        out_specs=[
            pl.BlockSpec((rb, D), lambda i: (i, 0)),
            pl.BlockSpec((rb, D), lambda i: (i, 0)),
        ],
        out_shape=(
            jax.ShapeDtypeStruct((N, D), jnp.float32),
            jax.ShapeDtypeStruct((N, D), jnp.float32),
        ),
    )(x)
    nflat = nb[:, 0]

    edge_fn = _make_edge_kernel(e_full, nb0, nb1)
    acc, denp = edge_fn(xn, nflat, src_full, dst_full, beta16)
    den_t = denp.reshape(NC, NP)[:, :N].T

    out = pl.pallas_call(
        _final_body,
        grid=(grid_r,),
        in_specs=[
            pl.BlockSpec((1, rb, D), lambda i: (0, i, 0)),
            pl.BlockSpec((1, rb, D), lambda i: (1, i, 0)),
            pl.BlockSpec((rb, NC), lambda i: (i, 0)),
        ],
        out_specs=pl.BlockSpec((rb, D), lambda i: (i, 0)),
        out_shape=jax.ShapeDtypeStruct((N, D), jnp.float32),
    )(acc, acc, den_t)
    return out


# speedup vs baseline: 1.0072x; 1.0072x over previous
import functools

import jax
import jax.numpy as jnp
from jax import lax
from jax.experimental import pallas as pl
from jax.experimental.pallas import tpu as pltpu
from jax.experimental.pallas import tpu_sc as plsc

N = 10000
D = 128
NC = 2
NS = 16
NW = NC * NS
B = 32
NP = 10240
ROWS_PER_TILE = NP // NS
DEN_R = NP // 128


def _norm_body(x_ref, xn_ref, nb_ref):
    xb = x_ref[...]
    ss = jnp.sum(xb * xb, axis=1, keepdims=True)
    nrm = jnp.maximum(jnp.sqrt(ss), 1e-12)
    xn_ref[...] = xb / nrm
    nb_ref[...] = jnp.broadcast_to(nrm, xb.shape)


def _final_body(a0_ref, a1_ref, dt_ref, o_ref):
    den = jnp.sum(dt_ref[...], axis=1, keepdims=True)
    s = a0_ref[0] + a1_ref[0]
    o_ref[...] = jnp.maximum(s / jnp.maximum(den, 1e-16), 0.0)


def _make_edge_kernel(e_full, nb0, nb1):
    mesh = plsc.VectorSubcoreMesh(
        core_axis_name="c", subcore_axis_name="s", num_cores=NC, num_subcores=NS
    )

    @functools.partial(
        pl.kernel,
        out_type=(
            jax.ShapeDtypeStruct((NC, NP, D), jnp.float32),
            jax.ShapeDtypeStruct((NC, DEN_R, 128), jnp.float32),
        ),
        mesh=mesh,
        compiler_params=pltpu.CompilerParams(needs_layout_passes=False),
        scratch_types=[
            pltpu.VMEM((N,), jnp.float32),
            pltpu.VMEM((DEN_R, 128), jnp.float32),
            pltpu.VMEM((3, B), jnp.int32),
            pltpu.VMEM((3, B), jnp.int32),
            pltpu.VMEM((3, B), jnp.int32),
            pltpu.VMEM((3, B, D), jnp.float32),
            pltpu.VMEM((3, B, D), jnp.float32),
            pltpu.VMEM((16,), jnp.float32),
            pltpu.VMEM((DEN_R,), jnp.int32),
            pltpu.SemaphoreType.DMA((3,)),
            pltpu.SemaphoreType.DMA((3,)),
            pltpu.SemaphoreType.DMA((3,)),
            pltpu.VMEM_SHARED((NP, D), jnp.float32),
            pltpu.VMEM_SHARED((DEN_R, 128), jnp.float32),
        ],
    )
    def edge_kernel(xn_hbm, n_hbm, src_hbm, dst_hbm, beta_hbm,
                    acc_out, den_out,
                    ntab, denom_v, sidx, didx, sdidx, srows, drows,
                    beta_v, iden, isem, gsem, ssem, acc_sh, dden_sh):
        cid = lax.axis_index("c")
        sid = lax.axis_index("s")
        n_my = jnp.where(cid == 0, nb0, nb1)
        tile_start = jnp.where(cid == 0, sid * (nb0 * B),
                               NS * (nb0 * B) + sid * (nb1 * B))

        pltpu.sync_copy(n_hbm, ntab)
        pltpu.sync_copy(beta_hbm, beta_v)
        bvec = beta_v[pl.ds(0, 16)]

        zero16 = jnp.zeros((16,), jnp.float32)

        def zero_den(i, carry):
            for s in range(8):
                denom_v[i, pl.ds(s * 16, 16)] = zero16
            return carry

        lax.fori_loop(0, DEN_R, zero_den, 0)

        def zero_sr(i, carry):
            for s in range(8):
                srows[0, i, pl.ds(s * 16, 16)] = zero16
            return carry

        lax.fori_loop(0, B, zero_sr, 0)

        lane = lax.iota(jnp.int32, 16)
        for q in range(DEN_R // 16):
            iden[pl.ds(q * 16, 16)] = lane + q * 16

        row0 = sid * ROWS_PER_TILE
        for q in range(ROWS_PER_TILE // B):
            pltpu.sync_copy(srows.at[0], acc_sh.at[pl.ds(row0 + q * B, B)])

        @pl.when(sid == 0)
        def _():
            pltpu.sync_copy(srows.at[0], dden_sh.at[pl.ds(0, B)])
            pltpu.sync_copy(srows.at[0], dden_sh.at[pl.ds(B, B)])
            pltpu.sync_copy(srows.at[0, pl.ds(0, DEN_R - 2 * B)],
                            dden_sh.at[pl.ds(2 * B, DEN_R - 2 * B)])

        plsc.subcore_barrier()

        def idx_start(slot, blk_id):
            base = tile_start + blk_id * B
            pltpu.async_copy(src_hbm.at[pl.ds(base, B)], sidx.at[slot],
                             isem.at[slot])
            pltpu.async_copy(dst_hbm.at[pl.ds(base, B)], didx.at[slot],
                             isem.at[slot])

        def idx_wait(slot, blk_id):
            base = tile_start + blk_id * B
            pltpu.make_async_copy(src_hbm.at[pl.ds(base, B)], sidx.at[slot],
                                  isem.at[slot]).wait()
            pltpu.make_async_copy(dst_hbm.at[pl.ds(base, B)], didx.at[slot],
                                  isem.at[slot]).wait()

        def gather_start(slot):
            pltpu.async_copy(xn_hbm.at[sidx.at[slot]], srows.at[slot],
                             gsem.at[slot])
            pltpu.async_copy(xn_hbm.at[didx.at[slot]], drows.at[slot],
                             gsem.at[slot])

        def gather_wait(slot):
            pltpu.make_async_copy(xn_hbm.at[sidx.at[slot]], srows.at[slot],
                                  gsem.at[slot]).wait()
            pltpu.make_async_copy(xn_hbm.at[didx.at[slot]], drows.at[slot],
                                  gsem.at[slot]).wait()

        def scatter_start(slot):
            pltpu.async_copy(srows.at[slot], acc_sh.at[sdidx.at[slot]],
                             ssem.at[slot], add=True)

        def scatter_wait(slot):
            pltpu.make_async_copy(srows.at[slot], acc_sh.at[sdidx.at[slot]],
                                  ssem.at[slot]).wait()

        def compute_block(slot, g):
            base = tile_start + g * B

            def grp(k, c2):
                sl = pl.ds(k * 16, 16)
                si = sidx[slot, sl]
                di = didx[slot, sl]
                d16 = jnp.zeros((16,), jnp.float32)
                for j in range(16):
                    e = k * 16 + j
                    acc = (srows[slot, e, pl.ds(0, 16)]
                           * drows[slot, e, pl.ds(0, 16)])
                    for s in range(1, 8):
                        fsl = pl.ds(s * 16, 16)
                        acc = acc + srows[slot, e, fsl] * drows[slot, e, fsl]
                    d16 = jnp.where(lane == j, jnp.sum(acc), d16)
                gid = base + k * 16 + lane
                w = jnp.where(gid < e_full, jnp.exp(bvec * d16), 0.0)
                plsc.addupdate_scatter(denom_v, [di // 128, di % 128], w)
                n_s = plsc.load_gather(ntab, [si])
                wn = w * n_s
                sdidx[slot, sl] = di
                for j in range(16):
                    e = k * 16 + j
                    wj = wn[j]
                    for s in range(8):
                        fsl = pl.ds(s * 16, 16)
                        srows[slot, e, fsl] = srows[slot, e, fsl] * wj
                return c2

            lax.fori_loop(0, B // 16, grp, 0)

        idx_start(0, 0)
        idx_start(1, 1)
        idx_wait(0, 0)
        gather_start(0)

        def blk(g, carry):
            slot = lax.rem(g, 3)
            sn = lax.rem(g + 1, 3)
            sp = lax.rem(g + 2, 3)

            @pl.when(g >= 2)
            def _():
                scatter_wait(sn)

            @pl.when(g + 1 < n_my)
            def _():
                idx_wait(sn, g + 1)
                gather_start(sn)

            @pl.when(g + 2 < n_my)
            def _():
                idx_start(sp, g + 2)

            gather_wait(slot)
            compute_block(slot, g)
            scatter_start(slot)
            return carry

        lax.fori_loop(0, n_my, blk, 0)
        scatter_wait(lax.rem(n_my - 2, 3))
        scatter_wait(lax.rem(n_my - 1, 3))

        pltpu.sync_copy(denom_v, dden_sh.at[iden], add=True)
        plsc.subcore_barrier()
        pltpu.sync_copy(acc_sh.at[pl.ds(row0, ROWS_PER_TILE)],
                        acc_out.at[cid, pl.ds(row0, ROWS_PER_TILE)])

        @pl.when(sid == 0)
        def _():
            pltpu.sync_copy(dden_sh, den_out.at[cid])

    return edge_kernel


def kernel(x, edge_index, beta):
    src = edge_index[0].astype(jnp.int32)
    dst = edge_index[1].astype(jnp.int32)
    loop = jnp.arange(N, dtype=jnp.int32)
    e_full = src.shape[0] + N
    nb_tot = -(-e_full // (NS * B))
    nb0 = int(nb_tot * 0.50)
    nb1 = nb_tot - nb0
    e_pad = NS * B * (nb0 + nb1)
    pad = e_pad - e_full
    src_full = jnp.concatenate([src, loop, jnp.zeros((pad,), jnp.int32)])
    dst_full = jnp.concatenate([dst, loop, jnp.zeros((pad,), jnp.int32)])
    beta16 = jnp.broadcast_to(beta.astype(jnp.float32), (16,))

    grid_r = 10
    rb = N // grid_r
    xn, nb = pl.pallas_call(
        _norm_body,
        grid=(grid_r,),
        in_specs=[pl.BlockSpec((rb, D), lambda i: (i, 0))],
        out_specs=[
            pl.BlockSpec((rb, D), lambda i: (i, 0)),
            pl.BlockSpec((rb, D), lambda i: (i, 0)),
        ],
        out_shape=(
            jax.ShapeDtypeStruct((N, D), jnp.float32),
            jax.ShapeDtypeStruct((N, D), jnp.float32),
        ),
    )(x)
    nflat = nb[:, 0]

    edge_fn = _make_edge_kernel(e_full, nb0, nb1)
    acc, denp = edge_fn(xn, nflat, src_full, dst_full, beta16)
    den_t = denp.reshape(NC, NP)[:, :N].T

    out = pl.pallas_call(
        _final_body,
        grid=(grid_r,),
        in_specs=[
            pl.BlockSpec((1, rb, D), lambda i: (0, i, 0)),
            pl.BlockSpec((1, rb, D), lambda i: (1, i, 0)),
            pl.BlockSpec((rb, NC), lambda i: (i, 0)),
        ],
        out_specs=pl.BlockSpec((rb, D), lambda i: (i, 0)),
        out_shape=jax.ShapeDtypeStruct((N, D), jnp.float32),
    )(acc, acc, den_t)
    return out
